# Initial kernel scaffold; baseline (speedup 1.0000x reference)
import jax
import jax.numpy as jnp
from jax.experimental import pallas as pl


def kernel(all_rois, gt_boxes):
    raise NotImplementedError("write your pallas kernel here")


# speedup vs baseline: 2.9079x; 2.9079x over previous
import jax
import jax.numpy as jnp
from jax import lax
from jax.experimental import pallas as pl

_BG_PID = 5532.0
_BATCH = 128
_NUM_FG = 32
_FG_THRESH = 0.5
_BG_HI = 0.5
_BG_LO = 0.1
_N_PAD = 20480
_CHUNK = 2048
_NCHUNK = _N_PAD // _CHUNK
_ROWS = _N_PAD // 128
_STDS = (0.1, 0.1, 0.2, 0.2)
_HI = lax.Precision.HIGHEST


def _iou_kern(rois_ref, gtt_ref, maxs_ref, amax_ref):
    f32 = jnp.float32
    gtt = gtt_ref[...]
    qx1 = gtt[0:1, :]
    qy1 = gtt[1:2, :]
    qx2 = gtt[2:3, :]
    qy2 = gtt[3:4, :]
    qarea = (qx2 - qx1 + 1.0) * (qy2 - qy1 + 1.0)

    def body(c, carry):
        blk = rois_ref[pl.ds(c * _CHUNK, _CHUNK), :]
        bx1 = blk[:, 1:2]
        by1 = blk[:, 2:3]
        bx2 = blk[:, 3:4]
        by2 = blk[:, 4:5]
        barea = (bx2 - bx1 + 1.0) * (by2 - by1 + 1.0)
        iw = jnp.maximum(jnp.minimum(bx2, qx2) - jnp.maximum(bx1, qx1) + 1.0, 0.0)
        ih = jnp.maximum(jnp.minimum(by2, qy2) - jnp.maximum(by1, qy1) + 1.0, 0.0)
        inter = iw * ih
        ua = barea + qarea - inter
        ov = inter / ua
        maxs_ref[pl.ds(c * _CHUNK, _CHUNK), :] = jnp.max(ov, axis=1, keepdims=True)
        amax_ref[pl.ds(c * _CHUNK, _CHUNK), :] = (
            jnp.argmax(ov, axis=1, keepdims=True).astype(f32))
        return carry

    lax.fori_loop(0, _NCHUNK, body, 0)


def _slot_kern(maxs_ref, slot_ref):
    f32 = jnp.float32
    maxs = maxs_ref[...]
    fgm = (maxs >= _FG_THRESH).astype(f32)
    bgm = jnp.logical_and(maxs < _BG_HI, maxs >= _BG_LO).astype(f32)

    ii = lax.broadcasted_iota(jnp.int32, (128, 128), 0)
    jj = lax.broadcasted_iota(jnp.int32, (128, 128), 1)
    tri_incl = (ii <= jj).astype(f32)
    incl_fg = jnp.dot(fgm, tri_incl, precision=_HI)
    incl_bg = jnp.dot(bgm, tri_incl, precision=_HI)

    si = lax.broadcasted_iota(jnp.int32, (_ROWS, _ROWS), 0)
    sj = lax.broadcasted_iota(jnp.int32, (_ROWS, _ROWS), 1)
    strict = (sj < si).astype(f32)
    pref_fg = jnp.dot(strict, incl_fg[:, 127:128], precision=_HI)
    pref_bg = jnp.dot(strict, incl_bg[:, 127:128], precision=_HI)

    t_fg = pref_fg + incl_fg - 1.0
    t_bg = pref_bg + incl_bg - 1.0
    slot_ref[...] = jnp.where(
        jnp.logical_and(fgm > 0.0, t_fg < float(_NUM_FG)), t_fg,
        jnp.where(jnp.logical_and(bgm > 0.0, t_bg < float(_BATCH - _NUM_FG)),
                  t_bg + float(_NUM_FG), -1.0))


def _gather_kern(slot_ref, amaxr_ref, rois_ref, gt_ref,
                 rois_out, lab_out, pid_out, bt_out, biw_out, bow_out):
    f32 = jnp.float32
    kcol = lax.broadcasted_iota(jnp.int32, (_BATCH, 1), 0).astype(f32)
    slot_row = slot_ref[...]
    ohT = (slot_row == kcol).astype(f32)
    filled = jnp.sum(ohT, axis=1, keepdims=True)
    akept = jnp.sum(ohT * amaxr_ref[...], axis=1, keepdims=True)
    kept = jnp.dot(ohT, rois_ref[...], precision=_HI)
    nf = 1.0 - filled
    kept = kept + nf * rois_ref[0:1, :]
    akept = akept + nf * amaxr_ref[0:1, 0:1]

    ohg = (akept == lax.broadcasted_iota(jnp.int32, (1, 128), 1).astype(f32)).astype(f32)
    assigned = jnp.dot(ohg, gt_ref[...], precision=_HI)

    rx1 = kept[:, 1:2]
    ry1 = kept[:, 2:3]
    rx2 = kept[:, 3:4]
    ry2 = kept[:, 4:5]
    gx1 = assigned[:, 0:1]
    gy1 = assigned[:, 1:2]
    gx2 = assigned[:, 2:3]
    gy2 = assigned[:, 3:4]
    ew = rx2 - rx1 + 1.0
    eh = ry2 - ry1 + 1.0
    ecx = rx1 + 0.5 * ew
    ecy = ry1 + 0.5 * eh
    gw = gx2 - gx1 + 1.0
    gh = gy2 - gy1 + 1.0
    gcx = gx1 + 0.5 * gw
    gcy = gy1 + 0.5 * gh
    dx = (gcx - ecx) / ew / _STDS[0]
    dy = (gcy - ecy) / eh / _STDS[1]
    dw = jnp.log(gw / ew) / _STDS[2]
    dh = jnp.log(gh / eh) / _STDS[3]
    data = jnp.concatenate([dx, dy, dw, dh], axis=1)

    srow = lax.broadcasted_iota(jnp.int32, (_BATCH, 1), 0).astype(f32)
    isfg_slot = srow < float(_NUM_FG)
    labels = jnp.where(isfg_slot, assigned[:, 4:5], 0.0)
    pids = jnp.where(isfg_slot, assigned[:, 5:6], _BG_PID)
    fgw = (labels > 0.0).astype(f32)
    clsr = jnp.round(labels)
    m0 = (clsr == 0.0).astype(f32)
    m1 = (clsr == 1.0).astype(f32)
    d = data * fgw
    ones4 = jnp.ones((_BATCH, 4), dtype=f32)
    bt_out[...] = jnp.concatenate([d * m0, d * m1], axis=1)
    biw_out[...] = jnp.concatenate([ones4 * (fgw * m0), ones4 * (fgw * m1)], axis=1)
    bow_out[...] = jnp.concatenate([ones4 * (fgw * m0), ones4 * (fgw * m1)], axis=1)
    rois_out[...] = kept[:, 0:5]
    lab_out[...] = labels.astype(jnp.int32)
    pid_out[...] = pids.astype(jnp.int32)


@jax.jit
def kernel(all_rois, gt_boxes):
    G = gt_boxes.shape[0]
    gt_rois = jnp.concatenate(
        [jnp.zeros((G, 1), jnp.float32), gt_boxes[:, :4]], axis=1)
    rois_ext = jnp.concatenate([all_rois, gt_rois], axis=0)
    pad = _N_PAD - rois_ext.shape[0]
    pad_rows = jnp.full((pad, 5), -1e9, dtype=jnp.float32)
    rois_p = jnp.concatenate([rois_ext, pad_rows], axis=0)
    rois_p8 = jnp.concatenate(
        [rois_p, jnp.zeros((_N_PAD, 3), jnp.float32)], axis=1)
    gtt = gt_boxes[:, :4].T
    gtt8 = jnp.concatenate([gtt, jnp.zeros((4, G), jnp.float32)], axis=0)

    maxs, amax = pl.pallas_call(
        _iou_kern,
        out_shape=(
            jax.ShapeDtypeStruct((_N_PAD, 1), jnp.float32),
            jax.ShapeDtypeStruct((_N_PAD, 1), jnp.float32),
        ),
    )(rois_p8, gtt8)

    slot = pl.pallas_call(
        _slot_kern,
        out_shape=jax.ShapeDtypeStruct((_ROWS, 128), jnp.float32),
    )(maxs.reshape(_ROWS, 128))

    outs = pl.pallas_call(
        _gather_kern,
        out_shape=(
            jax.ShapeDtypeStruct((_BATCH, 5), jnp.float32),
            jax.ShapeDtypeStruct((_BATCH, 1), jnp.int32),
            jax.ShapeDtypeStruct((_BATCH, 1), jnp.int32),
            jax.ShapeDtypeStruct((_BATCH, 8), jnp.float32),
            jax.ShapeDtypeStruct((_BATCH, 8), jnp.float32),
            jax.ShapeDtypeStruct((_BATCH, 8), jnp.float32),
        ),
    )(slot.reshape(1, _N_PAD), amax.reshape(1, _N_PAD), rois_p8, gt_boxes)
    rois, lab, pid, bt, biw, bow = outs
    return (rois, lab.reshape(_BATCH), pid.reshape(_BATCH), bt, biw, bow)


# speedup vs baseline: 4.6187x; 1.5883x over previous
import jax
import jax.numpy as jnp
from jax import lax
from jax.experimental import pallas as pl

_BG_PID = 5532.0
_BATCH = 128
_NUM_FG = 32
_FG_THRESH = 0.5
_BG_HI = 0.5
_BG_LO = 0.1
_N_PAD = 20480
_CHUNK = 2048
_NCHUNK = _N_PAD // _CHUNK
_ROWS = _N_PAD // 128
_STDS = (0.1, 0.1, 0.2, 0.2)
_HI = lax.Precision.HIGHEST


def _iou_kern(roisT_ref, gt_ref, maxs_ref, amax_ref):
    f32 = jnp.float32
    qx1 = gt_ref[:, 0:1]
    qy1 = gt_ref[:, 1:2]
    qx2 = gt_ref[:, 2:3]
    qy2 = gt_ref[:, 3:4]
    qarea = (qx2 - qx1 + 1.0) * (qy2 - qy1 + 1.0)

    for c in range(_NCHUNK):
        sl = pl.ds(c * _CHUNK, _CHUNK)
        bx1 = roisT_ref[1:2, sl]
        by1 = roisT_ref[2:3, sl]
        bx2 = roisT_ref[3:4, sl]
        by2 = roisT_ref[4:5, sl]
        barea = (bx2 - bx1 + 1.0) * (by2 - by1 + 1.0)
        iw = jnp.maximum(jnp.minimum(bx2, qx2) - jnp.maximum(bx1, qx1) + 1.0, 0.0)
        ih = jnp.maximum(jnp.minimum(by2, qy2) - jnp.maximum(by1, qy1) + 1.0, 0.0)
        inter = iw * ih
        ua = barea + qarea - inter
        ov = inter / ua
        maxs_ref[c:c + 1, :] = jnp.max(ov, axis=0, keepdims=True)
        amax_ref[c:c + 1, :] = jnp.argmax(ov, axis=0, keepdims=True).astype(f32)


def _slot_kern(maxs_ref, slot_ref):
    f32 = jnp.float32
    maxs = maxs_ref[...]
    fgm = (maxs >= _FG_THRESH).astype(f32)
    bgm = jnp.logical_and(maxs < _BG_HI, maxs >= _BG_LO).astype(f32)

    ii = lax.broadcasted_iota(jnp.int32, (128, 128), 0)
    jj = lax.broadcasted_iota(jnp.int32, (128, 128), 1)
    tri_incl = (ii <= jj).astype(f32)
    incl_fg = jnp.dot(fgm, tri_incl, precision=_HI)
    incl_bg = jnp.dot(bgm, tri_incl, precision=_HI)

    si = lax.broadcasted_iota(jnp.int32, (_ROWS, _ROWS), 0)
    sj = lax.broadcasted_iota(jnp.int32, (_ROWS, _ROWS), 1)
    strict = (sj < si).astype(f32)
    pref_fg = jnp.dot(strict, incl_fg[:, 127:128], precision=_HI)
    pref_bg = jnp.dot(strict, incl_bg[:, 127:128], precision=_HI)

    t_fg = pref_fg + incl_fg - 1.0
    t_bg = pref_bg + incl_bg - 1.0
    slot_ref[...] = jnp.where(
        jnp.logical_and(fgm > 0.0, t_fg < float(_NUM_FG)), t_fg,
        jnp.where(jnp.logical_and(bgm > 0.0, t_bg < float(_BATCH - _NUM_FG)),
                  t_bg + float(_NUM_FG), -1.0))


def _gather_kern(slot_ref, amaxr_ref, rois_ref, gt_ref,
                 rois_out, lab_out, pid_out, bt_out, biw_out, bow_out):
    f32 = jnp.float32
    kcol = lax.broadcasted_iota(jnp.int32, (_BATCH, 1), 0).astype(f32)
    slot_row = slot_ref[...]
    ohT = (slot_row == kcol).astype(f32)
    filled = jnp.sum(ohT, axis=1, keepdims=True)
    akept = jnp.sum(ohT * amaxr_ref[...], axis=1, keepdims=True)
    kept = jnp.dot(ohT, rois_ref[...], precision=_HI)
    nf = 1.0 - filled
    kept = kept + nf * rois_ref[0:1, :]
    akept = akept + nf * amaxr_ref[0:1, 0:1]

    ohg = (akept == lax.broadcasted_iota(jnp.int32, (1, 128), 1).astype(f32)).astype(f32)
    assigned = jnp.dot(ohg, gt_ref[...], precision=_HI)

    rx1 = kept[:, 1:2]
    ry1 = kept[:, 2:3]
    rx2 = kept[:, 3:4]
    ry2 = kept[:, 4:5]
    gx1 = assigned[:, 0:1]
    gy1 = assigned[:, 1:2]
    gx2 = assigned[:, 2:3]
    gy2 = assigned[:, 3:4]
    ew = rx2 - rx1 + 1.0
    eh = ry2 - ry1 + 1.0
    ecx = rx1 + 0.5 * ew
    ecy = ry1 + 0.5 * eh
    gw = gx2 - gx1 + 1.0
    gh = gy2 - gy1 + 1.0
    gcx = gx1 + 0.5 * gw
    gcy = gy1 + 0.5 * gh
    dx = (gcx - ecx) / ew / _STDS[0]
    dy = (gcy - ecy) / eh / _STDS[1]
    dw = jnp.log(gw / ew) / _STDS[2]
    dh = jnp.log(gh / eh) / _STDS[3]
    data = jnp.concatenate([dx, dy, dw, dh], axis=1)

    srow = lax.broadcasted_iota(jnp.int32, (_BATCH, 1), 0).astype(f32)
    isfg_slot = srow < float(_NUM_FG)
    labels = jnp.where(isfg_slot, assigned[:, 4:5], 0.0)
    pids = jnp.where(isfg_slot, assigned[:, 5:6], _BG_PID)
    fgw = (labels > 0.0).astype(f32)
    clsr = jnp.round(labels)
    m0 = (clsr == 0.0).astype(f32)
    m1 = (clsr == 1.0).astype(f32)
    d = data * fgw
    ones4 = jnp.ones((_BATCH, 4), dtype=f32)
    bt_out[...] = jnp.concatenate([d * m0, d * m1], axis=1)
    biw_out[...] = jnp.concatenate([ones4 * (fgw * m0), ones4 * (fgw * m1)], axis=1)
    bow_out[...] = jnp.concatenate([ones4 * (fgw * m0), ones4 * (fgw * m1)], axis=1)
    rois_out[...] = kept[:, 0:5]
    lab_out[...] = labels.astype(jnp.int32)
    pid_out[...] = pids.astype(jnp.int32)


@jax.jit
def kernel(all_rois, gt_boxes):
    G = gt_boxes.shape[0]
    gt_rois = jnp.concatenate(
        [jnp.zeros((G, 1), jnp.float32), gt_boxes[:, :4]], axis=1)
    rois_ext = jnp.concatenate([all_rois, gt_rois], axis=0)
    pad = _N_PAD - rois_ext.shape[0]
    pad_rows = jnp.full((pad, 5), -1e9, dtype=jnp.float32)
    rois_p = jnp.concatenate([rois_ext, pad_rows], axis=0)
    rois_p8 = jnp.concatenate(
        [rois_p, jnp.zeros((_N_PAD, 3), jnp.float32)], axis=1)
    roisT8 = jnp.concatenate(
        [rois_p.T, jnp.zeros((3, _N_PAD), jnp.float32)], axis=0)

    maxs, amax = pl.pallas_call(
        _iou_kern,
        out_shape=(
            jax.ShapeDtypeStruct((_NCHUNK, _CHUNK), jnp.float32),
            jax.ShapeDtypeStruct((_NCHUNK, _CHUNK), jnp.float32),
        ),
    )(roisT8, gt_boxes)

    slot = pl.pallas_call(
        _slot_kern,
        out_shape=jax.ShapeDtypeStruct((_ROWS, 128), jnp.float32),
    )(maxs.reshape(_ROWS, 128))

    outs = pl.pallas_call(
        _gather_kern,
        out_shape=(
            jax.ShapeDtypeStruct((_BATCH, 5), jnp.float32),
            jax.ShapeDtypeStruct((_BATCH, 1), jnp.int32),
            jax.ShapeDtypeStruct((_BATCH, 1), jnp.int32),
            jax.ShapeDtypeStruct((_BATCH, 8), jnp.float32),
            jax.ShapeDtypeStruct((_BATCH, 8), jnp.float32),
            jax.ShapeDtypeStruct((_BATCH, 8), jnp.float32),
        ),
    )(slot.reshape(1, _N_PAD), amax.reshape(1, _N_PAD), rois_p8, gt_boxes)
    rois, lab, pid, bt, biw, bow = outs
    return (rois, lab.reshape(_BATCH), pid.reshape(_BATCH), bt, biw, bow)


# speedup vs baseline: 6.3126x; 1.3667x over previous
import jax
import jax.numpy as jnp
from jax import lax
from jax.experimental import pallas as pl

_BG_PID = 5532.0
_BATCH = 128
_NUM_FG = 32
_FG_THRESH = 0.5
_BG_HI = 0.5
_BG_LO = 0.1
_N_PAD = 20480
_CHUNK = 2048
_NCHUNK = _N_PAD // _CHUNK
_ROWS = _N_PAD // 128
_STDS = (0.1, 0.1, 0.2, 0.2)
_HI = lax.Precision.HIGHEST


def _iou_kern(roisT_ref, gt_ref, maxs_ref, amax_ref):
    f32 = jnp.float32
    qx1 = gt_ref[:, 0:1]
    qy1 = gt_ref[:, 1:2]
    qx2 = gt_ref[:, 2:3]
    qy2 = gt_ref[:, 3:4]
    qarea = (qx2 - qx1 + 1.0) * (qy2 - qy1 + 1.0)

    for c in range(_NCHUNK):
        sl = pl.ds(c * _CHUNK, _CHUNK)
        bx1 = roisT_ref[1:2, sl]
        by1 = roisT_ref[2:3, sl]
        bx2 = roisT_ref[3:4, sl]
        by2 = roisT_ref[4:5, sl]
        barea = (bx2 - bx1 + 1.0) * (by2 - by1 + 1.0)
        iw = jnp.maximum(jnp.minimum(bx2, qx2) - jnp.maximum(bx1, qx1) + 1.0, 0.0)
        ih = jnp.maximum(jnp.minimum(by2, qy2) - jnp.maximum(by1, qy1) + 1.0, 0.0)
        inter = iw * ih
        ua = barea + qarea - inter
        ov = inter / ua
        maxs_ref[c:c + 1, :] = jnp.max(ov, axis=0, keepdims=True)
        amax_ref[c:c + 1, :] = jnp.argmax(ov, axis=0, keepdims=True).astype(f32)


def _slot_kern(maxs_ref, slot_ref):
    f32 = jnp.float32
    maxs = maxs_ref[...]
    fgm = (maxs >= _FG_THRESH).astype(f32)
    bgm = jnp.logical_and(maxs < _BG_HI, maxs >= _BG_LO).astype(f32)

    ii = lax.broadcasted_iota(jnp.int32, (128, 128), 0)
    jj = lax.broadcasted_iota(jnp.int32, (128, 128), 1)
    tri_incl = (ii <= jj).astype(f32)
    incl_fg = jnp.dot(fgm, tri_incl, precision=_HI)
    incl_bg = jnp.dot(bgm, tri_incl, precision=_HI)

    si = lax.broadcasted_iota(jnp.int32, (_ROWS, _ROWS), 0)
    sj = lax.broadcasted_iota(jnp.int32, (_ROWS, _ROWS), 1)
    strict = (sj < si).astype(f32)
    pref_fg = jnp.dot(strict, incl_fg[:, 127:128], precision=_HI)
    pref_bg = jnp.dot(strict, incl_bg[:, 127:128], precision=_HI)

    t_fg = pref_fg + incl_fg - 1.0
    t_bg = pref_bg + incl_bg - 1.0
    slot_ref[...] = jnp.where(
        jnp.logical_and(fgm > 0.0, t_fg < float(_NUM_FG)), t_fg,
        jnp.where(jnp.logical_and(bgm > 0.0, t_bg < float(_BATCH - _NUM_FG)),
                  t_bg + float(_NUM_FG), -1.0))


def _gather_kern(slot_ref, xs_ref, gt_ref,
                 rois_out, lab_out, pid_out, bt_out, biw_out, bow_out):
    f32 = jnp.float32
    kcol = lax.broadcasted_iota(jnp.int32, (_BATCH, 1), 0).astype(f32)
    slot_row = slot_ref[...]
    ohT = (slot_row == kcol).astype(jnp.bfloat16)
    acc = jnp.dot(ohT, xs_ref[...], preferred_element_type=f32)
    kept = acc[:, 0:8] + acc[:, 8:16] + acc[:, 16:24]
    akept = acc[:, 24:25]
    filled = acc[:, 25:26]
    r0acc = xs_ref[0:1, :].astype(f32)
    r0 = r0acc[:, 0:8] + r0acc[:, 8:16] + r0acc[:, 16:24]
    nf = 1.0 - filled
    kept = kept + nf * r0
    akept = akept + nf * r0acc[:, 24:25]

    ohg = (akept == lax.broadcasted_iota(jnp.int32, (1, 128), 1).astype(f32)).astype(f32)
    assigned = jnp.dot(ohg, gt_ref[...], precision=_HI)

    rx1 = kept[:, 1:2]
    ry1 = kept[:, 2:3]
    rx2 = kept[:, 3:4]
    ry2 = kept[:, 4:5]
    gx1 = assigned[:, 0:1]
    gy1 = assigned[:, 1:2]
    gx2 = assigned[:, 2:3]
    gy2 = assigned[:, 3:4]
    ew = rx2 - rx1 + 1.0
    eh = ry2 - ry1 + 1.0
    ecx = rx1 + 0.5 * ew
    ecy = ry1 + 0.5 * eh
    gw = gx2 - gx1 + 1.0
    gh = gy2 - gy1 + 1.0
    gcx = gx1 + 0.5 * gw
    gcy = gy1 + 0.5 * gh
    dx = (gcx - ecx) / ew / _STDS[0]
    dy = (gcy - ecy) / eh / _STDS[1]
    dw = jnp.log(gw / ew) / _STDS[2]
    dh = jnp.log(gh / eh) / _STDS[3]
    data = jnp.concatenate([dx, dy, dw, dh], axis=1)

    srow = lax.broadcasted_iota(jnp.int32, (_BATCH, 1), 0).astype(f32)
    isfg_slot = srow < float(_NUM_FG)
    labels = jnp.where(isfg_slot, assigned[:, 4:5], 0.0)
    pids = jnp.where(isfg_slot, assigned[:, 5:6], _BG_PID)
    fgw = (labels > 0.0).astype(f32)
    clsr = jnp.round(labels)
    m0 = (clsr == 0.0).astype(f32)
    m1 = (clsr == 1.0).astype(f32)
    d = data * fgw
    ones4 = jnp.ones((_BATCH, 4), dtype=f32)
    bt_out[...] = jnp.concatenate([d * m0, d * m1], axis=1)
    biw_out[...] = jnp.concatenate([ones4 * (fgw * m0), ones4 * (fgw * m1)], axis=1)
    bow_out[...] = jnp.concatenate([ones4 * (fgw * m0), ones4 * (fgw * m1)], axis=1)
    rois_out[...] = kept[:, 0:5]
    lab_out[...] = labels.astype(jnp.int32)
    pid_out[...] = pids.astype(jnp.int32)


@jax.jit
def kernel(all_rois, gt_boxes):
    G = gt_boxes.shape[0]
    gt_rois = jnp.concatenate(
        [jnp.zeros((G, 1), jnp.float32), gt_boxes[:, :4]], axis=1)
    rois_ext = jnp.concatenate([all_rois, gt_rois], axis=0)
    pad = _N_PAD - rois_ext.shape[0]
    pad_rows = jnp.full((pad, 5), -1e9, dtype=jnp.float32)
    rois_p = jnp.concatenate([rois_ext, pad_rows], axis=0)
    rois_p8 = jnp.concatenate(
        [rois_p, jnp.zeros((_N_PAD, 3), jnp.float32)], axis=1)
    roisT8 = jnp.concatenate(
        [rois_p.T, jnp.zeros((3, _N_PAD), jnp.float32)], axis=0)

    maxs, amax = pl.pallas_call(
        _iou_kern,
        out_shape=(
            jax.ShapeDtypeStruct((_NCHUNK, _CHUNK), jnp.float32),
            jax.ShapeDtypeStruct((_NCHUNK, _CHUNK), jnp.float32),
        ),
    )(roisT8, gt_boxes)

    slot = pl.pallas_call(
        _slot_kern,
        out_shape=jax.ShapeDtypeStruct((_ROWS, 128), jnp.float32),
    )(maxs.reshape(_ROWS, 128))

    mask = jnp.uint32(0xFFFF0000)
    xu = lax.bitcast_convert_type(rois_p8, jnp.uint32)
    hi = lax.bitcast_convert_type(xu & mask, jnp.float32)
    lo_raw = rois_p8 - hi
    lo = lax.bitcast_convert_type(
        lax.bitcast_convert_type(lo_raw, jnp.uint32) & mask, jnp.float32)
    lo2 = lo_raw - lo
    xsplit = jnp.concatenate(
        [hi, lo, lo2, amax.reshape(_N_PAD, 1),
         jnp.ones((_N_PAD, 1), jnp.float32),
         jnp.zeros((_N_PAD, 6), jnp.float32)], axis=1).astype(jnp.bfloat16)

    outs = pl.pallas_call(
        _gather_kern,
        out_shape=(
            jax.ShapeDtypeStruct((_BATCH, 5), jnp.float32),
            jax.ShapeDtypeStruct((_BATCH, 1), jnp.int32),
            jax.ShapeDtypeStruct((_BATCH, 1), jnp.int32),
            jax.ShapeDtypeStruct((_BATCH, 8), jnp.float32),
            jax.ShapeDtypeStruct((_BATCH, 8), jnp.float32),
            jax.ShapeDtypeStruct((_BATCH, 8), jnp.float32),
        ),
    )(slot.reshape(1, _N_PAD), xsplit, gt_boxes)
    rois, lab, pid, bt, biw, bow = outs
    return (rois, lab.reshape(_BATCH), pid.reshape(_BATCH), bt, biw, bow)
